# Initial kernel scaffold; baseline (speedup 1.0000x reference)
import jax
import jax.numpy as jnp
from jax.experimental import pallas as pl


def kernel(dst_feat, feat_0, feat_1, feat_2, feat_3, edge_index_1, edge_index_2, edge_index_3, WT_w_0, WT_b_0, WT_w_1, WT_b_1, WT_w_2, WT_b_2, WT_w_3, WT_b_3, attn_l_1, attn_r_1, attn_l_2, attn_r_2, attn_l_3, attn_r_3, sem_W1, sem_b1, sem_W2):
    raise NotImplementedError("write your pallas kernel here")


# speedup vs baseline: 19.4007x; 19.4007x over previous
import functools

import jax
import jax.numpy as jnp
from jax import lax
from jax.experimental import pallas as pl
from jax.experimental.pallas import tpu as pltpu
from jax.experimental.pallas import tpu_sc as plsc

N = 10000
D = 128
E = 320000
R = 3
NC = 2
NS = 16
NW = NC * NS
EP = E // NW
C1 = 2000
C2 = 400
IB = 80
NIB = C2 // IB
RPT = N // NS
NB = 5
BN = N // NB


def _stage_a_body(dst_ref, f1_ref, f2_ref, f3_ref,
                  w0_ref, b0_ref, w1_ref, b1_ref, w2_ref, b2_ref, w3_ref, b3_ref,
                  al_ref, ar_ref,
                  nf1_ref, nf2_ref, nf3_ref, el_ref, er_ref):
    h = jnp.dot(dst_ref[...], w0_ref[...],
                preferred_element_type=jnp.float32) + b0_ref[...]
    fs = (f1_ref, f2_ref, f3_ref)
    ws = (w1_ref, w2_ref, w3_ref)
    bs = (b1_ref, b2_ref, b3_ref)
    nfs = (nf1_ref, nf2_ref, nf3_ref)
    els = []
    ers = []
    for r in range(R):
        nf = jnp.dot(fs[r][...], ws[r][...],
                     preferred_element_type=jnp.float32) + bs[r][...]
        nfs[r][...] = nf
        els.append(jnp.sum(nf * al_ref[r:r + 1, :], axis=1))
        ers.append(jnp.sum(h * ar_ref[r:r + 1, :], axis=1))
    el_ref[...] = jnp.stack(els, axis=0)
    er_ref[...] = jnp.stack(ers, axis=0)


def _stage_a(dst_feat, f1, f2, f3, W0, b0, W1, b1, W2, b2, W3, b3, al, ar):
    return pl.pallas_call(
        _stage_a_body,
        out_shape=[jax.ShapeDtypeStruct((N, D), jnp.float32),
                   jax.ShapeDtypeStruct((N, D), jnp.float32),
                   jax.ShapeDtypeStruct((N, D), jnp.float32),
                   jax.ShapeDtypeStruct((R, N), jnp.float32),
                   jax.ShapeDtypeStruct((R, N), jnp.float32)],
    )(dst_feat, f1, f2, f3, W0, b0, W1, b1, W2, b2, W3, b3, al, ar)


def _phase1_body(el_hbm, er_hbm, src_hbm, dst_hbm,
                 ea_hbm, sp_hbm,
                 el_v, er_v, zero_v, srcc, dstc, idx80, eac,
                 s_sh0, s_sh1, s_sh2):
    ci = lax.axis_index("c")
    sid = lax.axis_index("s")
    wid = sid * NC + ci
    s_shs = (s_sh0, s_sh1, s_sh2)

    @pl.when(sid == 0)
    def _():
        def _zero(k, _):
            zero_v[pl.ds(k * 16, 16)] = jnp.zeros((16,), jnp.float32)
            return 0
        lax.fori_loop(0, N // 16, _zero, 0)
        for r in range(R):
            pltpu.sync_copy(zero_v, s_shs[r])
    plsc.subcore_barrier()

    for r in range(R):
        pltpu.sync_copy(el_hbm.at[pl.ds(r * N, N)], el_v)
        pltpu.sync_copy(er_hbm.at[pl.ds(r * N, N)], er_v)

        def _mx(k, acc):
            return jnp.maximum(acc, el_v[pl.ds(k * 16, 16)])
        acc = lax.fori_loop(0, N // 16, _mx,
                            jnp.full((16,), -jnp.inf, jnp.float32))
        elmax = acc[0]
        for k in range(1, 16):
            elmax = jnp.maximum(elmax, acc[k])

        def _chunk(ch, _):
            base = r * E + wid * EP + ch * C1
            pltpu.sync_copy(src_hbm.at[pl.ds(base, C1)], srcc)
            pltpu.sync_copy(dst_hbm.at[pl.ds(base, C1)], dstc)

            def _vec(i, _):
                s16 = srcc[pl.ds(i * 16, 16)]
                d16 = dstc[pl.ds(i * 16, 16)]
                elg = plsc.load_gather(el_v, [s16])
                erg = plsc.load_gather(er_v, [d16])
                x = elg + erg
                e = jnp.where(x > 0.0, x, 0.01 * x)
                m = jnp.maximum(elmax + erg, 0.0)
                eac[pl.ds(i * 16, 16)] = jnp.exp(e - m)
                return 0
            lax.fori_loop(0, C1 // 16, _vec, 0)

            for j in range(C1 // IB):
                pltpu.sync_copy(dst_hbm.at[pl.ds(base + j * IB, IB)], idx80)
                pltpu.sync_copy(eac.at[pl.ds(j * IB, IB)],
                                s_shs[r].at[idx80], add=True)
            pltpu.sync_copy(eac, ea_hbm.at[pl.ds(base, C1)])
            return 0
        lax.fori_loop(0, EP // C1, _chunk, 0)

    plsc.subcore_barrier()

    @pl.when(sid == 0)
    def _():
        for r in range(R):
            pltpu.sync_copy(s_shs[r], el_v)
            pltpu.sync_copy(el_v, sp_hbm.at[pl.ds((ci * R + r) * N, N)])


def _phase1(el, er, src, dst):
    mesh = plsc.VectorSubcoreMesh(core_axis_name="c", subcore_axis_name="s")
    f = functools.partial(
        pl.kernel,
        out_type=[jax.ShapeDtypeStruct((R * E,), jnp.float32),
                  jax.ShapeDtypeStruct((NC * R * N,), jnp.float32)],
        mesh=mesh,
        scratch_types=[pltpu.VMEM((N,), jnp.float32),
                       pltpu.VMEM((N,), jnp.float32),
                       pltpu.VMEM((N,), jnp.float32),
                       pltpu.VMEM((C1,), jnp.int32),
                       pltpu.VMEM((C1,), jnp.int32),
                       pltpu.VMEM((IB,), jnp.int32),
                       pltpu.VMEM((C1,), jnp.float32),
                       pltpu.VMEM_SHARED((N,), jnp.float32),
                       pltpu.VMEM_SHARED((N,), jnp.float32),
                       pltpu.VMEM_SHARED((N,), jnp.float32)],
        compiler_params=pltpu.CompilerParams(needs_layout_passes=False),
    )(_phase1_body)
    return f(el, er, src, dst)


def _stage_b_body(sp_ref, rinv_ref):
    s = jnp.sum(sp_ref[...], axis=0)
    rinv_ref[...] = 1.0 / jnp.maximum(s, 1e-30)


def _stage_b(sp):
    return pl.pallas_call(
        _stage_b_body,
        out_shape=jax.ShapeDtypeStruct((R, N), jnp.float32),
    )(sp)


def _phase2_body(nf1_hbm, nf2_hbm, nf3_hbm, ea_hbm, rinv_hbm,
                 src_hbm, dst_hbm,
                 out_hbm,
                 rinv_v, sidx, didx, dstc, eac, rows, sem, out_sh):
    ci = lax.axis_index("c")
    sid = lax.axis_index("s")
    wid = sid * NC + ci
    nfs = (nf1_hbm, nf2_hbm, nf3_hbm)
    for r in range(R):
        def _zr(i, _):
            for j in range(D // 16):
                rows[i, pl.ds(j * 16, 16)] = jnp.zeros((16,), jnp.float32)
            return 0
        lax.fori_loop(0, IB, _zr, 0)

        @pl.when(sid < NS - 6)
        def _():
            for q in range(12):
                pltpu.sync_copy(rows, out_sh.at[pl.ds(sid * 1000 + q * IB,
                                                      IB)])
            pltpu.sync_copy(rows.at[pl.ds(0, 1000 - 12 * IB)],
                            out_sh.at[pl.ds(sid * 1000 + 12 * IB,
                                            1000 - 12 * IB)])

        pltpu.sync_copy(rinv_hbm.at[pl.ds(r * N, N)], rinv_v)
        plsc.subcore_barrier()

        def _chunk(ch, _):
            base = r * E + wid * EP + ch * C2
            pltpu.sync_copy(dst_hbm.at[pl.ds(base, C2)], dstc)
            pltpu.sync_copy(ea_hbm.at[pl.ds(base, C2)], eac)

            def _a(i, _):
                d16 = dstc[pl.ds(i * 16, 16)]
                eac[pl.ds(i * 16, 16)] = (eac[pl.ds(i * 16, 16)] *
                                          plsc.load_gather(rinv_v, [d16]))
                return 0
            lax.fori_loop(0, C2 // 16, _a, 0)

            for j in range(NIB):
                pltpu.sync_copy(src_hbm.at[pl.ds(base + j * IB, IB)], sidx)
                pltpu.sync_copy(nfs[r].at[sidx], rows)

                def _s(g, _):
                    a16 = eac[pl.ds(j * IB + g * 16, 16)]
                    for k in range(16):
                        a_s = a16[k]
                        for jj in range(D // 16):
                            rows[g * 16 + k, pl.ds(jj * 16, 16)] = (
                                rows[g * 16 + k, pl.ds(jj * 16, 16)] * a_s)
                    return 0
                lax.fori_loop(0, IB // 16, _s, 0)

                pltpu.sync_copy(dst_hbm.at[pl.ds(base + j * IB, IB)], didx)
                pltpu.sync_copy(rows, out_sh.at[didx], add=True)
            return 0
        lax.fori_loop(0, EP // C2, _chunk, 0)
        plsc.subcore_barrier()

        @pl.when(sid < NS - 6)
        def _():
            for q in range(13):
                size = IB if q < 12 else 1000 - 12 * IB
                off = sid * 1000 + q * IB
                pltpu.sync_copy(out_sh.at[pl.ds(off, size)],
                                rows.at[pl.ds(0, size)])
                pltpu.sync_copy(rows.at[pl.ds(0, size)],
                                out_hbm.at[ci, r, pl.ds(off, size)])
        plsc.subcore_barrier()


def _phase2(nf1, nf2, nf3, ea, rinv, src, dst):
    mesh = plsc.VectorSubcoreMesh(core_axis_name="c", subcore_axis_name="s")
    f = functools.partial(
        pl.kernel,
        out_type=jax.ShapeDtypeStruct((NC, R, N, D), jnp.float32),
        mesh=mesh,
        scratch_types=[pltpu.VMEM((N,), jnp.float32),
                       pltpu.VMEM((IB,), jnp.int32),
                       pltpu.VMEM((IB,), jnp.int32),
                       pltpu.VMEM((C2,), jnp.int32),
                       pltpu.VMEM((C2,), jnp.float32),
                       pltpu.VMEM((IB, D), jnp.float32),
                       pltpu.SemaphoreType.DMA,
                       pltpu.VMEM_SHARED((N, D), jnp.float32)],
        compiler_params=pltpu.CompilerParams(needs_layout_passes=False),
    )(_phase2_body)
    return f(nf1, nf2, nf3, ea, rinv, src, dst)


def _stage_c1_body(p_ref, w1_ref, b1_ref, w2_ref, zm_ref, aw_ref, wsum_ref):
    i = pl.program_id(0)

    @pl.when(i == 0)
    def _():
        for r in range(R):
            wsum_ref[r] = 0.0

    for r in range(R):
        p = p_ref[0, r] + p_ref[1, r]
        z = jnp.where(p > 0.0, p, jnp.exp(p) - 1.0)
        zm_ref[r] = z
        t = jnp.tanh(jnp.dot(z, w1_ref[...],
                             preferred_element_type=jnp.float32) + b1_ref[...])
        tw = jnp.sum(t * w2_ref[...], axis=1)
        wsum_ref[r] = wsum_ref[r] + jnp.sum(tw)

    lanes = lax.broadcasted_iota(jnp.int32, (8, 128), 1)
    w0 = wsum_ref[0] * (1.0 / N)
    w1 = wsum_ref[1] * (1.0 / N)
    w2 = wsum_ref[2] * (1.0 / N)
    v = jnp.where(lanes == 0, w0,
                  jnp.where(lanes == 1, w1,
                            jnp.where(lanes == 2, w2, -jnp.inf)))
    e = jnp.exp(v - jnp.max(v))
    aw_ref[...] = e * (1.0 / jnp.sum(e[0:1, :]))


def _stage_c1(out_part, sem_W1, sem_b1, sem_W2):
    return pl.pallas_call(
        _stage_c1_body,
        grid=(NB,),
        in_specs=[pl.BlockSpec((NC, R, BN, D), lambda i: (0, 0, i, 0)),
                  pl.BlockSpec((D, D), lambda i: (0, 0)),
                  pl.BlockSpec((1, D), lambda i: (0, 0)),
                  pl.BlockSpec((1, D), lambda i: (0, 0))],
        out_specs=[pl.BlockSpec((R, BN, D), lambda i: (0, i, 0)),
                   pl.BlockSpec((8, 128), lambda i: (0, 0))],
        out_shape=[jax.ShapeDtypeStruct((R, N, D), jnp.float32),
                   jax.ShapeDtypeStruct((8, 128), jnp.float32)],
        scratch_shapes=[pltpu.SMEM((R,), jnp.float32)],
    )(out_part, sem_W1, sem_b1, sem_W2)


def _stage_c2_body(zm_ref, aw_ref, z_ref):
    z_ref[...] = (zm_ref[0] * aw_ref[0] + zm_ref[1] * aw_ref[1]
                  + zm_ref[2] * aw_ref[2])


def _stage_c2(zm, aw):
    return pl.pallas_call(
        _stage_c2_body,
        grid=(NB,),
        in_specs=[pl.BlockSpec((R, BN, D), lambda i: (0, i, 0)),
                  pl.BlockSpec(memory_space=pltpu.SMEM)],
        out_specs=pl.BlockSpec((BN, D), lambda i: (i, 0)),
        out_shape=jax.ShapeDtypeStruct((N, D), jnp.float32),
    )(zm, aw)


def kernel(dst_feat, feat_0, feat_1, feat_2, feat_3,
           edge_index_1, edge_index_2, edge_index_3,
           WT_w_0, WT_b_0, WT_w_1, WT_b_1, WT_w_2, WT_b_2, WT_w_3, WT_b_3,
           attn_l_1, attn_r_1, attn_l_2, attn_r_2, attn_l_3, attn_r_3,
           sem_W1, sem_b1, sem_W2):
    src = jnp.concatenate([edge_index_1[0], edge_index_2[0],
                           edge_index_3[0]]).astype(jnp.int32)
    dst = jnp.concatenate([edge_index_1[1], edge_index_2[1],
                           edge_index_3[1]]).astype(jnp.int32)
    al = jnp.concatenate([attn_l_1, attn_l_2, attn_l_3], axis=0)
    ar = jnp.concatenate([attn_r_1, attn_r_2, attn_r_3], axis=0)
    nf1, nf2, nf3, el, er = _stage_a(
        dst_feat, feat_1, feat_2, feat_3,
        WT_w_0, WT_b_0.reshape(1, D), WT_w_1, WT_b_1.reshape(1, D),
        WT_w_2, WT_b_2.reshape(1, D), WT_w_3, WT_b_3.reshape(1, D), al, ar)
    ea, sp = _phase1(el.reshape(R * N), er.reshape(R * N), src, dst)
    rinv = _stage_b(sp.reshape(NC, R, N))
    out_part = _phase2(nf1, nf2, nf3, ea, rinv.reshape(R * N), src, dst)
    zm, aw8 = _stage_c1(out_part, sem_W1, sem_b1.reshape(1, D),
                        sem_W2.reshape(1, D))
    return _stage_c2(zm, aw8[0, :R])


# speedup vs baseline: 24.6458x; 1.2704x over previous
import functools

import jax
import jax.numpy as jnp
from jax import lax
from jax.experimental import pallas as pl
from jax.experimental.pallas import tpu as pltpu
from jax.experimental.pallas import tpu_sc as plsc

N = 10000
D = 128
E = 320000
R = 3
NC = 2
NS = 16
NW = NC * NS
EP = E // NW
C1 = 2000
C2 = 400
IB = 80
NIB = C2 // IB
RPT = N // NS
NB = 5
BN = N // NB


def _stage_a_body(dst_ref, f1_ref, f2_ref, f3_ref,
                  w0_ref, b0_ref, w1_ref, b1_ref, w2_ref, b2_ref, w3_ref, b3_ref,
                  al_ref, ar_ref,
                  nf1_ref, nf2_ref, nf3_ref, el_ref, er_ref):
    h = jnp.dot(dst_ref[...], w0_ref[...],
                preferred_element_type=jnp.float32) + b0_ref[...]
    fs = (f1_ref, f2_ref, f3_ref)
    ws = (w1_ref, w2_ref, w3_ref)
    bs = (b1_ref, b2_ref, b3_ref)
    nfs = (nf1_ref, nf2_ref, nf3_ref)
    els = []
    ers = []
    for r in range(R):
        nf = jnp.dot(fs[r][...], ws[r][...],
                     preferred_element_type=jnp.float32) + bs[r][...]
        nfs[r][...] = nf
        els.append(jnp.sum(nf * al_ref[r:r + 1, :], axis=1))
        ers.append(jnp.sum(h * ar_ref[r:r + 1, :], axis=1))
    el_ref[...] = jnp.stack(els, axis=0)
    er_ref[...] = jnp.stack(ers, axis=0)


def _stage_a(dst_feat, f1, f2, f3, W0, b0, W1, b1, W2, b2, W3, b3, al, ar):
    return pl.pallas_call(
        _stage_a_body,
        out_shape=[jax.ShapeDtypeStruct((N, D), jnp.float32),
                   jax.ShapeDtypeStruct((N, D), jnp.float32),
                   jax.ShapeDtypeStruct((N, D), jnp.float32),
                   jax.ShapeDtypeStruct((R, N), jnp.float32),
                   jax.ShapeDtypeStruct((R, N), jnp.float32)],
    )(dst_feat, f1, f2, f3, W0, b0, W1, b1, W2, b2, W3, b3, al, ar)


def _phase1_body(el_hbm, er_hbm, src_hbm, dst_hbm,
                 ea_hbm, sp_hbm,
                 el_v, er_v, zero_v, srcc, dstc, idx80, eac,
                 s_sh0, s_sh1, s_sh2):
    ci = lax.axis_index("c")
    sid = lax.axis_index("s")
    wid = sid * NC + ci
    s_shs = (s_sh0, s_sh1, s_sh2)

    @pl.when(sid == 0)
    def _():
        def _zero(k, _):
            zero_v[pl.ds(k * 16, 16)] = jnp.zeros((16,), jnp.float32)
            return 0
        lax.fori_loop(0, N // 16, _zero, 0)
        for r in range(R):
            pltpu.sync_copy(zero_v, s_shs[r])
    plsc.subcore_barrier()

    for r in range(R):
        pltpu.sync_copy(el_hbm.at[pl.ds(r * N, N)], el_v)
        pltpu.sync_copy(er_hbm.at[pl.ds(r * N, N)], er_v)

        def _mx(k, acc):
            return jnp.maximum(acc, el_v[pl.ds(k * 16, 16)])
        acc = lax.fori_loop(0, N // 16, _mx,
                            jnp.full((16,), -jnp.inf, jnp.float32))
        elmax = acc[0]
        for k in range(1, 16):
            elmax = jnp.maximum(elmax, acc[k])

        def _chunk(ch, _):
            base = r * E + wid * EP + ch * C1
            pltpu.sync_copy(src_hbm.at[pl.ds(base, C1)], srcc)
            pltpu.sync_copy(dst_hbm.at[pl.ds(base, C1)], dstc)

            def _vec(i, _):
                s16 = srcc[pl.ds(i * 16, 16)]
                d16 = dstc[pl.ds(i * 16, 16)]
                elg = plsc.load_gather(el_v, [s16])
                erg = plsc.load_gather(er_v, [d16])
                x = elg + erg
                e = jnp.where(x > 0.0, x, 0.01 * x)
                m = jnp.maximum(elmax + erg, 0.0)
                eac[pl.ds(i * 16, 16)] = jnp.exp(e - m)
                return 0
            lax.fori_loop(0, C1 // 16, _vec, 0)

            for j in range(C1 // IB):
                pltpu.sync_copy(dst_hbm.at[pl.ds(base + j * IB, IB)], idx80)
                pltpu.sync_copy(eac.at[pl.ds(j * IB, IB)],
                                s_shs[r].at[idx80], add=True)
            pltpu.sync_copy(eac, ea_hbm.at[pl.ds(base, C1)])
            return 0
        lax.fori_loop(0, EP // C1, _chunk, 0)

    plsc.subcore_barrier()

    @pl.when(sid == 0)
    def _():
        for r in range(R):
            pltpu.sync_copy(s_shs[r], el_v)
            pltpu.sync_copy(el_v, sp_hbm.at[pl.ds((ci * R + r) * N, N)])


def _phase1(el, er, src, dst):
    mesh = plsc.VectorSubcoreMesh(core_axis_name="c", subcore_axis_name="s")
    f = functools.partial(
        pl.kernel,
        out_type=[jax.ShapeDtypeStruct((R * E,), jnp.float32),
                  jax.ShapeDtypeStruct((NC * R * N,), jnp.float32)],
        mesh=mesh,
        scratch_types=[pltpu.VMEM((N,), jnp.float32),
                       pltpu.VMEM((N,), jnp.float32),
                       pltpu.VMEM((N,), jnp.float32),
                       pltpu.VMEM((C1,), jnp.int32),
                       pltpu.VMEM((C1,), jnp.int32),
                       pltpu.VMEM((IB,), jnp.int32),
                       pltpu.VMEM((C1,), jnp.float32),
                       pltpu.VMEM_SHARED((N,), jnp.float32),
                       pltpu.VMEM_SHARED((N,), jnp.float32),
                       pltpu.VMEM_SHARED((N,), jnp.float32)],
        compiler_params=pltpu.CompilerParams(needs_layout_passes=False),
    )(_phase1_body)
    return f(el, er, src, dst)


def _stage_b_body(sp_ref, rinv_ref):
    s = jnp.sum(sp_ref[...], axis=0)
    rinv_ref[...] = 1.0 / jnp.maximum(s, 1e-30)


def _stage_b(sp):
    return pl.pallas_call(
        _stage_b_body,
        out_shape=jax.ShapeDtypeStruct((R, N), jnp.float32),
    )(sp)


def _phase2_body(nf1_hbm, nf2_hbm, nf3_hbm, ea_hbm, rinv_hbm,
                 src_hbm, dst_hbm,
                 out_hbm,
                 rinv_v, sidx0, sidx1, didx, dstc, eac, rows0, rows1,
                 sem0, sem1, out_sh):
    sidxs = (sidx0, sidx1)
    rowss = (rows0, rows1)
    sems = (sem0, sem1)
    rows = rows0
    ci = lax.axis_index("c")
    sid = lax.axis_index("s")
    wid = sid * NC + ci
    nfs = (nf1_hbm, nf2_hbm, nf3_hbm)
    for r in range(R):
        def _zr(i, _):
            for j in range(D // 16):
                rows[i, pl.ds(j * 16, 16)] = jnp.zeros((16,), jnp.float32)
            return 0
        lax.fori_loop(0, IB, _zr, 0)

        @pl.when(sid < NS - 6)
        def _():
            for q in range(12):
                pltpu.sync_copy(rows, out_sh.at[pl.ds(sid * 1000 + q * IB,
                                                      IB)])
            pltpu.sync_copy(rows.at[pl.ds(0, 1000 - 12 * IB)],
                            out_sh.at[pl.ds(sid * 1000 + 12 * IB,
                                            1000 - 12 * IB)])

        pltpu.sync_copy(rinv_hbm.at[pl.ds(r * N, N)], rinv_v)
        plsc.subcore_barrier()

        def _chunk(ch, _):
            base = r * E + wid * EP + ch * C2
            pltpu.sync_copy(dst_hbm.at[pl.ds(base, C2)], dstc)
            pltpu.sync_copy(ea_hbm.at[pl.ds(base, C2)], eac)

            def _a(i, _):
                d16 = dstc[pl.ds(i * 16, 16)]
                eac[pl.ds(i * 16, 16)] = (eac[pl.ds(i * 16, 16)] *
                                          plsc.load_gather(rinv_v, [d16]))
                return 0
            lax.fori_loop(0, C2 // 16, _a, 0)

            pltpu.sync_copy(src_hbm.at[pl.ds(base, IB)], sidx0)
            desc = pltpu.async_copy(nfs[r].at[sidx0], rows0, sem0)
            for j in range(NIB):
                cur = rowss[j % 2]
                if j + 1 < NIB:
                    nxt = (j + 1) % 2
                    pltpu.sync_copy(src_hbm.at[pl.ds(base + (j + 1) * IB,
                                                     IB)], sidxs[nxt])
                    ndesc = pltpu.async_copy(nfs[r].at[sidxs[nxt]],
                                             rowss[nxt], sems[nxt])
                desc.wait()

                def _s(g, _):
                    a16 = eac[pl.ds(j * IB + g * 16, 16)]
                    for k in range(16):
                        a_s = a16[k]
                        for jj in range(D // 16):
                            cur[g * 16 + k, pl.ds(jj * 16, 16)] = (
                                cur[g * 16 + k, pl.ds(jj * 16, 16)] * a_s)
                    return 0
                lax.fori_loop(0, IB // 16, _s, 0)

                pltpu.sync_copy(dst_hbm.at[pl.ds(base + j * IB, IB)], didx)
                pltpu.sync_copy(cur, out_sh.at[didx], add=True)
                if j + 1 < NIB:
                    desc = ndesc
            return 0
        lax.fori_loop(0, EP // C2, _chunk, 0)
        plsc.subcore_barrier()

        @pl.when(sid < NS - 6)
        def _():
            for q in range(13):
                size = IB if q < 12 else 1000 - 12 * IB
                off = sid * 1000 + q * IB
                pltpu.sync_copy(out_sh.at[pl.ds(off, size)],
                                rows.at[pl.ds(0, size)])
                pltpu.sync_copy(rows.at[pl.ds(0, size)],
                                out_hbm.at[ci, r, pl.ds(off, size)])
        plsc.subcore_barrier()


def _phase2(nf1, nf2, nf3, ea, rinv, src, dst):
    mesh = plsc.VectorSubcoreMesh(core_axis_name="c", subcore_axis_name="s")
    f = functools.partial(
        pl.kernel,
        out_type=jax.ShapeDtypeStruct((NC, R, N, D), jnp.float32),
        mesh=mesh,
        scratch_types=[pltpu.VMEM((N,), jnp.float32),
                       pltpu.VMEM((IB,), jnp.int32),
                       pltpu.VMEM((IB,), jnp.int32),
                       pltpu.VMEM((IB,), jnp.int32),
                       pltpu.VMEM((C2,), jnp.int32),
                       pltpu.VMEM((C2,), jnp.float32),
                       pltpu.VMEM((IB, D), jnp.float32),
                       pltpu.VMEM((IB, D), jnp.float32),
                       pltpu.SemaphoreType.DMA,
                       pltpu.SemaphoreType.DMA,
                       pltpu.VMEM_SHARED((N, D), jnp.float32)],
        compiler_params=pltpu.CompilerParams(needs_layout_passes=False),
    )(_phase2_body)
    return f(nf1, nf2, nf3, ea, rinv, src, dst)


def _stage_c1_body(p_ref, w1_ref, b1_ref, w2_ref, zm_ref, aw_ref, wsum_ref):
    i = pl.program_id(0)

    @pl.when(i == 0)
    def _():
        for r in range(R):
            wsum_ref[r] = 0.0

    for r in range(R):
        p = p_ref[0, r] + p_ref[1, r]
        z = jnp.where(p > 0.0, p, jnp.exp(p) - 1.0)
        zm_ref[r] = z
        t = jnp.tanh(jnp.dot(z, w1_ref[...],
                             preferred_element_type=jnp.float32) + b1_ref[...])
        tw = jnp.sum(t * w2_ref[...], axis=1)
        wsum_ref[r] = wsum_ref[r] + jnp.sum(tw)

    lanes = lax.broadcasted_iota(jnp.int32, (8, 128), 1)
    w0 = wsum_ref[0] * (1.0 / N)
    w1 = wsum_ref[1] * (1.0 / N)
    w2 = wsum_ref[2] * (1.0 / N)
    v = jnp.where(lanes == 0, w0,
                  jnp.where(lanes == 1, w1,
                            jnp.where(lanes == 2, w2, -jnp.inf)))
    e = jnp.exp(v - jnp.max(v))
    aw_ref[...] = e * (1.0 / jnp.sum(e[0:1, :]))


def _stage_c1(out_part, sem_W1, sem_b1, sem_W2):
    return pl.pallas_call(
        _stage_c1_body,
        grid=(NB,),
        in_specs=[pl.BlockSpec((NC, R, BN, D), lambda i: (0, 0, i, 0)),
                  pl.BlockSpec((D, D), lambda i: (0, 0)),
                  pl.BlockSpec((1, D), lambda i: (0, 0)),
                  pl.BlockSpec((1, D), lambda i: (0, 0))],
        out_specs=[pl.BlockSpec((R, BN, D), lambda i: (0, i, 0)),
                   pl.BlockSpec((8, 128), lambda i: (0, 0))],
        out_shape=[jax.ShapeDtypeStruct((R, N, D), jnp.float32),
                   jax.ShapeDtypeStruct((8, 128), jnp.float32)],
        scratch_shapes=[pltpu.SMEM((R,), jnp.float32)],
    )(out_part, sem_W1, sem_b1, sem_W2)


def _stage_c2_body(zm_ref, aw_ref, z_ref):
    z_ref[...] = (zm_ref[0] * aw_ref[0] + zm_ref[1] * aw_ref[1]
                  + zm_ref[2] * aw_ref[2])


def _stage_c2(zm, aw):
    return pl.pallas_call(
        _stage_c2_body,
        grid=(NB,),
        in_specs=[pl.BlockSpec((R, BN, D), lambda i: (0, i, 0)),
                  pl.BlockSpec(memory_space=pltpu.SMEM)],
        out_specs=pl.BlockSpec((BN, D), lambda i: (i, 0)),
        out_shape=jax.ShapeDtypeStruct((N, D), jnp.float32),
    )(zm, aw)


def kernel(dst_feat, feat_0, feat_1, feat_2, feat_3,
           edge_index_1, edge_index_2, edge_index_3,
           WT_w_0, WT_b_0, WT_w_1, WT_b_1, WT_w_2, WT_b_2, WT_w_3, WT_b_3,
           attn_l_1, attn_r_1, attn_l_2, attn_r_2, attn_l_3, attn_r_3,
           sem_W1, sem_b1, sem_W2):
    src = jnp.concatenate([edge_index_1[0], edge_index_2[0],
                           edge_index_3[0]]).astype(jnp.int32)
    dst = jnp.concatenate([edge_index_1[1], edge_index_2[1],
                           edge_index_3[1]]).astype(jnp.int32)
    al = jnp.concatenate([attn_l_1, attn_l_2, attn_l_3], axis=0)
    ar = jnp.concatenate([attn_r_1, attn_r_2, attn_r_3], axis=0)
    nf1, nf2, nf3, el, er = _stage_a(
        dst_feat, feat_1, feat_2, feat_3,
        WT_w_0, WT_b_0.reshape(1, D), WT_w_1, WT_b_1.reshape(1, D),
        WT_w_2, WT_b_2.reshape(1, D), WT_w_3, WT_b_3.reshape(1, D), al, ar)
    ea, sp = _phase1(el.reshape(R * N), er.reshape(R * N), src, dst)
    rinv = _stage_b(sp.reshape(NC, R, N))
    out_part = _phase2(nf1, nf2, nf3, ea, rinv.reshape(R * N), src, dst)
    zm, aw8 = _stage_c1(out_part, sem_W1, sem_b1.reshape(1, D),
                        sem_W2.reshape(1, D))
    return _stage_c2(zm, aw8[0, :R])


# speedup vs baseline: 30.0687x; 1.2200x over previous
import functools

import jax
import jax.numpy as jnp
from jax import lax
from jax.experimental import pallas as pl
from jax.experimental.pallas import tpu as pltpu
from jax.experimental.pallas import tpu_sc as plsc

N = 10000
D = 128
E = 320000
R = 3
NC = 2
NS = 16
NW = NC * NS
EP = E // NW
C1 = 2000
C2 = 400
IB = 80
NIB = C2 // IB
RPT = N // NS
NB = 5
BN = N // NB


def _stage_a_body(dst_ref, f1_ref, f2_ref, f3_ref,
                  w0_ref, b0_ref, w1_ref, b1_ref, w2_ref, b2_ref, w3_ref, b3_ref,
                  al_ref, ar_ref,
                  nf1_ref, nf2_ref, nf3_ref, el_ref, er_ref):
    h = jnp.dot(dst_ref[...], w0_ref[...],
                preferred_element_type=jnp.float32) + b0_ref[...]
    fs = (f1_ref, f2_ref, f3_ref)
    ws = (w1_ref, w2_ref, w3_ref)
    bs = (b1_ref, b2_ref, b3_ref)
    nfs = (nf1_ref, nf2_ref, nf3_ref)
    els = []
    ers = []
    for r in range(R):
        nf = jnp.dot(fs[r][...], ws[r][...],
                     preferred_element_type=jnp.float32) + bs[r][...]
        nfs[r][...] = nf
        els.append(jnp.sum(nf * al_ref[r:r + 1, :], axis=1))
        ers.append(jnp.sum(h * ar_ref[r:r + 1, :], axis=1))
    el_ref[...] = jnp.stack(els, axis=0)
    er_ref[...] = jnp.stack(ers, axis=0)


def _stage_a(dst_feat, f1, f2, f3, W0, b0, W1, b1, W2, b2, W3, b3, al, ar):
    return pl.pallas_call(
        _stage_a_body,
        out_shape=[jax.ShapeDtypeStruct((N, D), jnp.float32),
                   jax.ShapeDtypeStruct((N, D), jnp.float32),
                   jax.ShapeDtypeStruct((N, D), jnp.float32),
                   jax.ShapeDtypeStruct((R, N), jnp.float32),
                   jax.ShapeDtypeStruct((R, N), jnp.float32)],
    )(dst_feat, f1, f2, f3, W0, b0, W1, b1, W2, b2, W3, b3, al, ar)


def _phase1_body(el_hbm, er_hbm, src_hbm, dst_hbm,
                 ea_hbm, sp_hbm,
                 el_v, er_v, zero_v, srcc, dstc, idx80, eac,
                 s_sh0, s_sh1, s_sh2):
    ci = lax.axis_index("c")
    sid = lax.axis_index("s")
    wid = sid * NC + ci
    s_shs = (s_sh0, s_sh1, s_sh2)

    @pl.when(sid == 0)
    def _():
        def _zero(k, _):
            zero_v[pl.ds(k * 16, 16)] = jnp.zeros((16,), jnp.float32)
            return 0
        lax.fori_loop(0, N // 16, _zero, 0)
        for r in range(R):
            pltpu.sync_copy(zero_v, s_shs[r])
    plsc.subcore_barrier()

    for r in range(R):
        pltpu.sync_copy(el_hbm.at[pl.ds(r * N, N)], el_v)
        pltpu.sync_copy(er_hbm.at[pl.ds(r * N, N)], er_v)

        def _mx(k, acc):
            return jnp.maximum(acc, el_v[pl.ds(k * 16, 16)])
        acc = lax.fori_loop(0, N // 16, _mx,
                            jnp.full((16,), -jnp.inf, jnp.float32))
        elmax = acc[0]
        for k in range(1, 16):
            elmax = jnp.maximum(elmax, acc[k])

        def _chunk(ch, _):
            base = r * E + wid * EP + ch * C1
            pltpu.sync_copy(src_hbm.at[pl.ds(base, C1)], srcc)
            pltpu.sync_copy(dst_hbm.at[pl.ds(base, C1)], dstc)

            def _vec(i, _):
                s16 = srcc[pl.ds(i * 16, 16)]
                d16 = dstc[pl.ds(i * 16, 16)]
                elg = plsc.load_gather(el_v, [s16])
                erg = plsc.load_gather(er_v, [d16])
                x = elg + erg
                e = jnp.where(x > 0.0, x, 0.01 * x)
                m = jnp.maximum(elmax + erg, 0.0)
                eac[pl.ds(i * 16, 16)] = jnp.exp(e - m)
                return 0
            lax.fori_loop(0, C1 // 16, _vec, 0)

            for j in range(C1 // IB):
                pltpu.sync_copy(dst_hbm.at[pl.ds(base + j * IB, IB)], idx80)
                pltpu.sync_copy(eac.at[pl.ds(j * IB, IB)],
                                s_shs[r].at[idx80], add=True)
            pltpu.sync_copy(eac, ea_hbm.at[pl.ds(base, C1)])
            return 0
        lax.fori_loop(0, EP // C1, _chunk, 0)

    plsc.subcore_barrier()

    @pl.when(sid == 0)
    def _():
        for r in range(R):
            pltpu.sync_copy(s_shs[r], el_v)
            pltpu.sync_copy(el_v, sp_hbm.at[pl.ds((ci * R + r) * N, N)])


def _phase1(el, er, src, dst):
    mesh = plsc.VectorSubcoreMesh(core_axis_name="c", subcore_axis_name="s")
    f = functools.partial(
        pl.kernel,
        out_type=[jax.ShapeDtypeStruct((R * E,), jnp.float32),
                  jax.ShapeDtypeStruct((NC * R * N,), jnp.float32)],
        mesh=mesh,
        scratch_types=[pltpu.VMEM((N,), jnp.float32),
                       pltpu.VMEM((N,), jnp.float32),
                       pltpu.VMEM((N,), jnp.float32),
                       pltpu.VMEM((C1,), jnp.int32),
                       pltpu.VMEM((C1,), jnp.int32),
                       pltpu.VMEM((IB,), jnp.int32),
                       pltpu.VMEM((C1,), jnp.float32),
                       pltpu.VMEM_SHARED((N,), jnp.float32),
                       pltpu.VMEM_SHARED((N,), jnp.float32),
                       pltpu.VMEM_SHARED((N,), jnp.float32)],
        compiler_params=pltpu.CompilerParams(needs_layout_passes=False),
    )(_phase1_body)
    return f(el, er, src, dst)


def _stage_b_body(sp_ref, rinv_ref):
    s = jnp.sum(sp_ref[...], axis=0)
    rinv_ref[...] = 1.0 / jnp.maximum(s, 1e-30)


def _stage_b(sp):
    return pl.pallas_call(
        _stage_b_body,
        out_shape=jax.ShapeDtypeStruct((R, N), jnp.float32),
    )(sp)


NSTG = 3


def _phase2_body(nf1_hbm, nf2_hbm, nf3_hbm, ea_hbm, rinv_hbm,
                 src_hbm, dst_hbm,
                 out_hbm,
                 rinv_v, dstc, eac,
                 sidx0, sidx1, sidx2, didx0, didx1, didx2,
                 rows0, rows1, rows2,
                 semg0, semg1, semg2, semd0, semd1, semd2,
                 sems0, sems1, sems2, out_sh):
    sidxs = (sidx0, sidx1, sidx2)
    didxs = (didx0, didx1, didx2)
    rowss = (rows0, rows1, rows2)
    semgs = (semg0, semg1, semg2)
    semds = (semd0, semd1, semd2)
    semss = (sems0, sems1, sems2)
    rows = rows0
    ci = lax.axis_index("c")
    sid = lax.axis_index("s")
    wid = sid * NC + ci
    nfs = (nf1_hbm, nf2_hbm, nf3_hbm)
    for r in range(R):
        def _zr(i, _):
            for j in range(D // 16):
                rows[i, pl.ds(j * 16, 16)] = jnp.zeros((16,), jnp.float32)
            return 0
        lax.fori_loop(0, IB, _zr, 0)

        @pl.when(sid < NS - 6)
        def _():
            for q in range(12):
                pltpu.sync_copy(rows, out_sh.at[pl.ds(sid * 1000 + q * IB,
                                                      IB)])
            pltpu.sync_copy(rows.at[pl.ds(0, 1000 - 12 * IB)],
                            out_sh.at[pl.ds(sid * 1000 + 12 * IB,
                                            1000 - 12 * IB)])

        pltpu.sync_copy(rinv_hbm.at[pl.ds(r * N, N)], rinv_v)
        plsc.subcore_barrier()

        def _chunk(ch, _):
            base = r * E + wid * EP + ch * C2
            pltpu.sync_copy(dst_hbm.at[pl.ds(base, C2)], dstc)
            pltpu.sync_copy(ea_hbm.at[pl.ds(base, C2)], eac)

            def _a(i, _):
                d16 = dstc[pl.ds(i * 16, 16)]
                eac[pl.ds(i * 16, 16)] = (eac[pl.ds(i * 16, 16)] *
                                          plsc.load_gather(rinv_v, [d16]))
                return 0
            lax.fori_loop(0, C2 // 16, _a, 0)

            pend_g, pend_d, pend_s = {}, {}, {}

            def _issue(j):
                s = j % NSTG
                pltpu.sync_copy(src_hbm.at[pl.ds(base + j * IB, IB)],
                                sidxs[s])
                pend_g[s] = pltpu.async_copy(nfs[r].at[sidxs[s]],
                                             rowss[s], semgs[s])
                pend_d[s] = pltpu.async_copy(
                    dst_hbm.at[pl.ds(base + j * IB, IB)], didxs[s], semds[s])

            _issue(0)
            for j in range(NIB):
                s = j % NSTG
                cur = rowss[s]
                if j + 1 < NIB:
                    s1 = (j + 1) % NSTG
                    if s1 in pend_s:
                        pend_s.pop(s1).wait()
                    _issue(j + 1)
                pend_g[s].wait()
                pend_d[s].wait()

                def _s(g, _):
                    a16 = eac[pl.ds(j * IB + g * 16, 16)]
                    for k in range(16):
                        a_s = a16[k]
                        for jj in range(D // 16):
                            cur[g * 16 + k, pl.ds(jj * 16, 16)] = (
                                cur[g * 16 + k, pl.ds(jj * 16, 16)] * a_s)
                    return 0
                lax.fori_loop(0, IB // 16, _s, 0)

                pend_s[s] = pltpu.async_copy(cur, out_sh.at[didxs[s]],
                                             semss[s], add=True)
            for s in list(pend_s):
                pend_s.pop(s).wait()
            return 0
        lax.fori_loop(0, EP // C2, _chunk, 0)
        plsc.subcore_barrier()

        @pl.when(sid < NS - 6)
        def _():
            for q in range(13):
                size = IB if q < 12 else 1000 - 12 * IB
                off = sid * 1000 + q * IB
                pltpu.sync_copy(out_sh.at[pl.ds(off, size)],
                                rows.at[pl.ds(0, size)])
                pltpu.sync_copy(rows.at[pl.ds(0, size)],
                                out_hbm.at[ci, r, pl.ds(off, size)])
        plsc.subcore_barrier()


def _phase2(nf1, nf2, nf3, ea, rinv, src, dst):
    mesh = plsc.VectorSubcoreMesh(core_axis_name="c", subcore_axis_name="s")
    f = functools.partial(
        pl.kernel,
        out_type=jax.ShapeDtypeStruct((NC, R, N, D), jnp.float32),
        mesh=mesh,
        scratch_types=([pltpu.VMEM((N,), jnp.float32),
                        pltpu.VMEM((C2,), jnp.int32),
                        pltpu.VMEM((C2,), jnp.float32)]
                       + [pltpu.VMEM((IB,), jnp.int32)] * 6
                       + [pltpu.VMEM((IB, D), jnp.float32)] * 3
                       + [pltpu.SemaphoreType.DMA] * 9
                       + [pltpu.VMEM_SHARED((N, D), jnp.float32)]),
        compiler_params=pltpu.CompilerParams(needs_layout_passes=False),
    )(_phase2_body)
    return f(nf1, nf2, nf3, ea, rinv, src, dst)


def _stage_c1_body(p_ref, w1_ref, b1_ref, w2_ref, zm_ref, aw_ref, wsum_ref):
    i = pl.program_id(0)

    @pl.when(i == 0)
    def _():
        for r in range(R):
            wsum_ref[r] = 0.0

    for r in range(R):
        p = p_ref[0, r] + p_ref[1, r]
        z = jnp.where(p > 0.0, p, jnp.exp(p) - 1.0)
        zm_ref[r] = z
        t = jnp.tanh(jnp.dot(z, w1_ref[...],
                             preferred_element_type=jnp.float32) + b1_ref[...])
        tw = jnp.sum(t * w2_ref[...], axis=1)
        wsum_ref[r] = wsum_ref[r] + jnp.sum(tw)

    lanes = lax.broadcasted_iota(jnp.int32, (8, 128), 1)
    w0 = wsum_ref[0] * (1.0 / N)
    w1 = wsum_ref[1] * (1.0 / N)
    w2 = wsum_ref[2] * (1.0 / N)
    v = jnp.where(lanes == 0, w0,
                  jnp.where(lanes == 1, w1,
                            jnp.where(lanes == 2, w2, -jnp.inf)))
    e = jnp.exp(v - jnp.max(v))
    aw_ref[...] = e * (1.0 / jnp.sum(e[0:1, :]))


def _stage_c1(out_part, sem_W1, sem_b1, sem_W2):
    return pl.pallas_call(
        _stage_c1_body,
        grid=(NB,),
        in_specs=[pl.BlockSpec((NC, R, BN, D), lambda i: (0, 0, i, 0)),
                  pl.BlockSpec((D, D), lambda i: (0, 0)),
                  pl.BlockSpec((1, D), lambda i: (0, 0)),
                  pl.BlockSpec((1, D), lambda i: (0, 0))],
        out_specs=[pl.BlockSpec((R, BN, D), lambda i: (0, i, 0)),
                   pl.BlockSpec((8, 128), lambda i: (0, 0))],
        out_shape=[jax.ShapeDtypeStruct((R, N, D), jnp.float32),
                   jax.ShapeDtypeStruct((8, 128), jnp.float32)],
        scratch_shapes=[pltpu.SMEM((R,), jnp.float32)],
    )(out_part, sem_W1, sem_b1, sem_W2)


def _stage_c2_body(zm_ref, aw_ref, z_ref):
    z_ref[...] = (zm_ref[0] * aw_ref[0] + zm_ref[1] * aw_ref[1]
                  + zm_ref[2] * aw_ref[2])


def _stage_c2(zm, aw):
    return pl.pallas_call(
        _stage_c2_body,
        grid=(NB,),
        in_specs=[pl.BlockSpec((R, BN, D), lambda i: (0, i, 0)),
                  pl.BlockSpec(memory_space=pltpu.SMEM)],
        out_specs=pl.BlockSpec((BN, D), lambda i: (i, 0)),
        out_shape=jax.ShapeDtypeStruct((N, D), jnp.float32),
    )(zm, aw)


def kernel(dst_feat, feat_0, feat_1, feat_2, feat_3,
           edge_index_1, edge_index_2, edge_index_3,
           WT_w_0, WT_b_0, WT_w_1, WT_b_1, WT_w_2, WT_b_2, WT_w_3, WT_b_3,
           attn_l_1, attn_r_1, attn_l_2, attn_r_2, attn_l_3, attn_r_3,
           sem_W1, sem_b1, sem_W2):
    src = jnp.concatenate([edge_index_1[0], edge_index_2[0],
                           edge_index_3[0]]).astype(jnp.int32)
    dst = jnp.concatenate([edge_index_1[1], edge_index_2[1],
                           edge_index_3[1]]).astype(jnp.int32)
    al = jnp.concatenate([attn_l_1, attn_l_2, attn_l_3], axis=0)
    ar = jnp.concatenate([attn_r_1, attn_r_2, attn_r_3], axis=0)
    nf1, nf2, nf3, el, er = _stage_a(
        dst_feat, feat_1, feat_2, feat_3,
        WT_w_0, WT_b_0.reshape(1, D), WT_w_1, WT_b_1.reshape(1, D),
        WT_w_2, WT_b_2.reshape(1, D), WT_w_3, WT_b_3.reshape(1, D), al, ar)
    ea, sp = _phase1(el.reshape(R * N), er.reshape(R * N), src, dst)
    rinv = _stage_b(sp.reshape(NC, R, N))
    out_part = _phase2(nf1, nf2, nf3, ea, rinv.reshape(R * N), src, dst)
    zm, aw8 = _stage_c1(out_part, sem_W1, sem_b1.reshape(1, D),
                        sem_W2.reshape(1, D))
    return _stage_c2(zm, aw8[0, :R])


# speedup vs baseline: 39.3056x; 1.3072x over previous
import functools

import jax
import jax.numpy as jnp
from jax import lax
from jax.experimental import pallas as pl
from jax.experimental.pallas import tpu as pltpu
from jax.experimental.pallas import tpu_sc as plsc

N = 10000
D = 128
E = 320000
R = 3
NC = 2
NS = 16
NW = NC * NS
EP = E // NW
C1 = 2000
C2 = 400
IB = 80
NIB = C2 // IB
RPT = N // NS
NB = 5
BN = N // NB


def _stage_a_body(dst_ref, f1_ref, f2_ref, f3_ref,
                  w0_ref, b0_ref, w1_ref, b1_ref, w2_ref, b2_ref, w3_ref, b3_ref,
                  al_ref, ar_ref,
                  nf1_ref, nf2_ref, nf3_ref, el_ref, er_ref):
    h = jnp.dot(dst_ref[...], w0_ref[...],
                preferred_element_type=jnp.float32) + b0_ref[...]
    fs = (f1_ref, f2_ref, f3_ref)
    ws = (w1_ref, w2_ref, w3_ref)
    bs = (b1_ref, b2_ref, b3_ref)
    nfs = (nf1_ref, nf2_ref, nf3_ref)
    els = []
    ers = []
    for r in range(R):
        nf = jnp.dot(fs[r][...], ws[r][...],
                     preferred_element_type=jnp.float32) + bs[r][...]
        nfs[r][...] = nf
        els.append(jnp.sum(nf * al_ref[r:r + 1, :], axis=1))
        ers.append(jnp.sum(h * ar_ref[r:r + 1, :], axis=1))
    el_ref[...] = jnp.stack(els, axis=0)
    er_ref[...] = jnp.stack(ers, axis=0)


def _stage_a(dst_feat, f1, f2, f3, W0, b0, W1, b1, W2, b2, W3, b3, al, ar):
    return pl.pallas_call(
        _stage_a_body,
        out_shape=[jax.ShapeDtypeStruct((N, D), jnp.float32),
                   jax.ShapeDtypeStruct((N, D), jnp.float32),
                   jax.ShapeDtypeStruct((N, D), jnp.float32),
                   jax.ShapeDtypeStruct((R, N), jnp.float32),
                   jax.ShapeDtypeStruct((R, N), jnp.float32)],
    )(dst_feat, f1, f2, f3, W0, b0, W1, b1, W2, b2, W3, b3, al, ar)


def _phase1_body(el_hbm, er_hbm, src_hbm, dst_hbm,
                 ea_hbm, sp_hbm,
                 el_v, er_v, zero_v, srcc, dstc,
                 idx0, idx1, idx2, idx3,
                 seml0, seml1, seml2, seml3,
                 semc0, semc1, semc2, semc3, eac,
                 s_sh0, s_sh1, s_sh2):
    idxs = (idx0, idx1, idx2, idx3)
    semls = (seml0, seml1, seml2, seml3)
    semcs = (semc0, semc1, semc2, semc3)
    ci = lax.axis_index("c")
    sid = lax.axis_index("s")
    wid = sid * NC + ci
    s_shs = (s_sh0, s_sh1, s_sh2)

    @pl.when(sid == 0)
    def _():
        def _zero(k, _):
            zero_v[pl.ds(k * 16, 16)] = jnp.zeros((16,), jnp.float32)
            return 0
        lax.fori_loop(0, N // 16, _zero, 0)
        for r in range(R):
            pltpu.sync_copy(zero_v, s_shs[r])
    plsc.subcore_barrier()

    for r in range(R):
        pltpu.sync_copy(el_hbm.at[pl.ds(r * N, N)], el_v)
        pltpu.sync_copy(er_hbm.at[pl.ds(r * N, N)], er_v)

        def _mx(k, acc):
            return jnp.maximum(acc, el_v[pl.ds(k * 16, 16)])
        acc = lax.fori_loop(0, N // 16, _mx,
                            jnp.full((16,), -jnp.inf, jnp.float32))
        elmax = acc[0]
        for k in range(1, 16):
            elmax = jnp.maximum(elmax, acc[k])

        def _chunk(ch, _):
            base = r * E + wid * EP + ch * C1
            pltpu.sync_copy(src_hbm.at[pl.ds(base, C1)], srcc)
            pltpu.sync_copy(dst_hbm.at[pl.ds(base, C1)], dstc)

            def _vec(i, _):
                s16 = srcc[pl.ds(i * 16, 16)]
                d16 = dstc[pl.ds(i * 16, 16)]
                elg = plsc.load_gather(el_v, [s16])
                erg = plsc.load_gather(er_v, [d16])
                x = elg + erg
                e = jnp.where(x > 0.0, x, 0.01 * x)
                m = jnp.maximum(elmax + erg, 0.0)
                eac[pl.ds(i * 16, 16)] = jnp.exp(e - m)
                return 0
            lax.fori_loop(0, C1 // 16, _vec, 0)

            pend_l, pend_sc = {}, {}

            def _load(j):
                s = j % 4
                pend_l[s] = pltpu.async_copy(
                    dst_hbm.at[pl.ds(base + j * IB, IB)], idxs[s], semls[s])

            _load(0)
            _load(1)
            for j in range(C1 // IB):
                s = j % 4
                if j + 2 < C1 // IB:
                    s2 = (j + 2) % 4
                    if s2 in pend_sc:
                        pend_sc.pop(s2).wait()
                    _load(j + 2)
                pend_l[s].wait()
                pend_sc[s] = pltpu.async_copy(
                    eac.at[pl.ds(j * IB, IB)], s_shs[r].at[idxs[s]],
                    semcs[s], add=True)
            for s in list(pend_sc):
                pend_sc.pop(s).wait()
            pltpu.sync_copy(eac, ea_hbm.at[pl.ds(base, C1)])
            return 0
        lax.fori_loop(0, EP // C1, _chunk, 0)

    plsc.subcore_barrier()

    @pl.when(sid == 0)
    def _():
        for r in range(R):
            pltpu.sync_copy(s_shs[r], el_v)
            pltpu.sync_copy(el_v, sp_hbm.at[pl.ds((ci * R + r) * N, N)])


def _phase1(el, er, src, dst):
    mesh = plsc.VectorSubcoreMesh(core_axis_name="c", subcore_axis_name="s")
    f = functools.partial(
        pl.kernel,
        out_type=[jax.ShapeDtypeStruct((R * E,), jnp.float32),
                  jax.ShapeDtypeStruct((NC * R * N,), jnp.float32)],
        mesh=mesh,
        scratch_types=([pltpu.VMEM((N,), jnp.float32),
                        pltpu.VMEM((N,), jnp.float32),
                        pltpu.VMEM((N,), jnp.float32),
                        pltpu.VMEM((C1,), jnp.int32),
                        pltpu.VMEM((C1,), jnp.int32)]
                       + [pltpu.VMEM((IB,), jnp.int32)] * 4
                       + [pltpu.SemaphoreType.DMA] * 8
                       + [pltpu.VMEM((C1,), jnp.float32),
                          pltpu.VMEM_SHARED((N,), jnp.float32),
                          pltpu.VMEM_SHARED((N,), jnp.float32),
                          pltpu.VMEM_SHARED((N,), jnp.float32)]),
        compiler_params=pltpu.CompilerParams(needs_layout_passes=False),
    )(_phase1_body)
    return f(el, er, src, dst)


def _stage_b_body(sp_ref, rinv_ref):
    s = jnp.sum(sp_ref[...], axis=0)
    rinv_ref[...] = 1.0 / jnp.maximum(s, 1e-30)


def _stage_b(sp):
    return pl.pallas_call(
        _stage_b_body,
        out_shape=jax.ShapeDtypeStruct((R, N), jnp.float32),
    )(sp)


NSTG = 3


def _phase2_body(nf1_hbm, nf2_hbm, nf3_hbm, ea_hbm, rinv_hbm,
                 src_hbm, dst_hbm,
                 out_hbm,
                 rinv_v, eac,
                 sidx0, sidx1, sidx2,
                 didx0, didx1, didx2, didx3,
                 rows0, rows1, rows2,
                 seml0, seml1, seml2,
                 semd0, semd1, semd2, semd3,
                 semg0, semg1, semg2,
                 sems0, sems1, sems2, out_sh):
    sidxs = (sidx0, sidx1, sidx2)
    didxs = (didx0, didx1, didx2, didx3)
    rowss = (rows0, rows1, rows2)
    semls = (seml0, seml1, seml2)
    semds = (semd0, semd1, semd2, semd3)
    semgs = (semg0, semg1, semg2)
    semss = (sems0, sems1, sems2)
    rows = rows0
    ci = lax.axis_index("c")
    sid = lax.axis_index("s")
    wid = sid * NC + ci
    nfs = (nf1_hbm, nf2_hbm, nf3_hbm)
    for r in range(R):
        def _zr(i, _):
            for j in range(D // 16):
                rows[i, pl.ds(j * 16, 16)] = jnp.zeros((16,), jnp.float32)
            return 0
        lax.fori_loop(0, IB, _zr, 0)

        @pl.when(sid < NS - 6)
        def _():
            for q in range(12):
                pltpu.sync_copy(rows, out_sh.at[pl.ds(sid * 1000 + q * IB,
                                                      IB)])
            pltpu.sync_copy(rows.at[pl.ds(0, 1000 - 12 * IB)],
                            out_sh.at[pl.ds(sid * 1000 + 12 * IB,
                                            1000 - 12 * IB)])

        pltpu.sync_copy(rinv_hbm.at[pl.ds(r * N, N)], rinv_v)
        plsc.subcore_barrier()

        def _chunk(ch, _):
            base = r * E + wid * EP + ch * C2
            pltpu.sync_copy(ea_hbm.at[pl.ds(base, C2)], eac)

            pend_li, pend_di, pend_g, pend_s = {}, {}, {}, {}

            def _load_idx(j):
                s = j % NSTG
                s4 = j % 4
                pend_li[s] = pltpu.async_copy(
                    src_hbm.at[pl.ds(base + j * IB, IB)], sidxs[s], semls[s])
                pend_di[s4] = pltpu.async_copy(
                    dst_hbm.at[pl.ds(base + j * IB, IB)], didxs[s4],
                    semds[s4])

            def _start_gather(j):
                s = j % NSTG
                pend_g[s] = pltpu.async_copy(nfs[r].at[sidxs[s]],
                                             rowss[s], semgs[s])

            _load_idx(0)
            pend_li[0].wait()
            _start_gather(0)
            if NIB > 1:
                _load_idx(1)
            for j in range(NIB):
                s = j % NSTG
                s4 = j % 4
                cur = rowss[s]
                if j + 1 < NIB:
                    s1 = (j + 1) % NSTG
                    if s1 in pend_s:
                        pend_s.pop(s1).wait()
                    pend_li[s1].wait()
                    _start_gather(j + 1)
                if j + 2 < NIB:
                    _load_idx(j + 2)
                pend_g[s].wait()
                pend_di[s4].wait()
                didx = didxs[s4]

                def _s(g, _):
                    d16 = didx[pl.ds(g * 16, 16)]
                    a16 = (eac[pl.ds(j * IB + g * 16, 16)] *
                           plsc.load_gather(rinv_v, [d16]))
                    for k in range(16):
                        a_s = a16[k]
                        for jj in range(D // 16):
                            cur[g * 16 + k, pl.ds(jj * 16, 16)] = (
                                cur[g * 16 + k, pl.ds(jj * 16, 16)] * a_s)
                    return 0
                lax.fori_loop(0, IB // 16, _s, 0)

                pend_s[s] = pltpu.async_copy(cur, out_sh.at[didxs[s4]],
                                             semss[s], add=True)
            for s in list(pend_s):
                pend_s.pop(s).wait()
            return 0
        lax.fori_loop(0, EP // C2, _chunk, 0)
        plsc.subcore_barrier()

        @pl.when(sid < NS - 6)
        def _():
            for q in range(13):
                size = IB if q < 12 else 1000 - 12 * IB
                off = sid * 1000 + q * IB
                pltpu.sync_copy(out_sh.at[pl.ds(off, size)],
                                rows.at[pl.ds(0, size)])
                pltpu.sync_copy(rows.at[pl.ds(0, size)],
                                out_hbm.at[ci, r, pl.ds(off, size)])
        plsc.subcore_barrier()


def _phase2(nf1, nf2, nf3, ea, rinv, src, dst):
    mesh = plsc.VectorSubcoreMesh(core_axis_name="c", subcore_axis_name="s")
    f = functools.partial(
        pl.kernel,
        out_type=jax.ShapeDtypeStruct((NC, R, N, D), jnp.float32),
        mesh=mesh,
        scratch_types=([pltpu.VMEM((N,), jnp.float32),
                        pltpu.VMEM((C2,), jnp.float32)]
                       + [pltpu.VMEM((IB,), jnp.int32)] * 7
                       + [pltpu.VMEM((IB, D), jnp.float32)] * 3
                       + [pltpu.SemaphoreType.DMA] * 13
                       + [pltpu.VMEM_SHARED((N, D), jnp.float32)]),
        compiler_params=pltpu.CompilerParams(needs_layout_passes=False),
    )(_phase2_body)
    return f(nf1, nf2, nf3, ea, rinv, src, dst)


def _stage_c1_body(p_ref, w1_ref, b1_ref, w2_ref, zm_ref, aw_ref, wsum_ref):
    i = pl.program_id(0)

    @pl.when(i == 0)
    def _():
        for r in range(R):
            wsum_ref[r] = 0.0

    for r in range(R):
        p = p_ref[0, r] + p_ref[1, r]
        z = jnp.where(p > 0.0, p, jnp.exp(p) - 1.0)
        zm_ref[r] = z
        t = jnp.tanh(jnp.dot(z, w1_ref[...],
                             preferred_element_type=jnp.float32) + b1_ref[...])
        tw = jnp.sum(t * w2_ref[...], axis=1)
        wsum_ref[r] = wsum_ref[r] + jnp.sum(tw)

    lanes = lax.broadcasted_iota(jnp.int32, (8, 128), 1)
    w0 = wsum_ref[0] * (1.0 / N)
    w1 = wsum_ref[1] * (1.0 / N)
    w2 = wsum_ref[2] * (1.0 / N)
    v = jnp.where(lanes == 0, w0,
                  jnp.where(lanes == 1, w1,
                            jnp.where(lanes == 2, w2, -jnp.inf)))
    e = jnp.exp(v - jnp.max(v))
    aw_ref[...] = e * (1.0 / jnp.sum(e[0:1, :]))


def _stage_c1(out_part, sem_W1, sem_b1, sem_W2):
    return pl.pallas_call(
        _stage_c1_body,
        grid=(NB,),
        in_specs=[pl.BlockSpec((NC, R, BN, D), lambda i: (0, 0, i, 0)),
                  pl.BlockSpec((D, D), lambda i: (0, 0)),
                  pl.BlockSpec((1, D), lambda i: (0, 0)),
                  pl.BlockSpec((1, D), lambda i: (0, 0))],
        out_specs=[pl.BlockSpec((R, BN, D), lambda i: (0, i, 0)),
                   pl.BlockSpec((8, 128), lambda i: (0, 0))],
        out_shape=[jax.ShapeDtypeStruct((R, N, D), jnp.float32),
                   jax.ShapeDtypeStruct((8, 128), jnp.float32)],
        scratch_shapes=[pltpu.SMEM((R,), jnp.float32)],
    )(out_part, sem_W1, sem_b1, sem_W2)


def _stage_c2_body(zm_ref, aw_ref, z_ref):
    z_ref[...] = (zm_ref[0] * aw_ref[0] + zm_ref[1] * aw_ref[1]
                  + zm_ref[2] * aw_ref[2])


def _stage_c2(zm, aw):
    return pl.pallas_call(
        _stage_c2_body,
        grid=(NB,),
        in_specs=[pl.BlockSpec((R, BN, D), lambda i: (0, i, 0)),
                  pl.BlockSpec(memory_space=pltpu.SMEM)],
        out_specs=pl.BlockSpec((BN, D), lambda i: (i, 0)),
        out_shape=jax.ShapeDtypeStruct((N, D), jnp.float32),
    )(zm, aw)


def kernel(dst_feat, feat_0, feat_1, feat_2, feat_3,
           edge_index_1, edge_index_2, edge_index_3,
           WT_w_0, WT_b_0, WT_w_1, WT_b_1, WT_w_2, WT_b_2, WT_w_3, WT_b_3,
           attn_l_1, attn_r_1, attn_l_2, attn_r_2, attn_l_3, attn_r_3,
           sem_W1, sem_b1, sem_W2):
    src = jnp.concatenate([edge_index_1[0], edge_index_2[0],
                           edge_index_3[0]]).astype(jnp.int32)
    dst = jnp.concatenate([edge_index_1[1], edge_index_2[1],
                           edge_index_3[1]]).astype(jnp.int32)
    al = jnp.concatenate([attn_l_1, attn_l_2, attn_l_3], axis=0)
    ar = jnp.concatenate([attn_r_1, attn_r_2, attn_r_3], axis=0)
    nf1, nf2, nf3, el, er = _stage_a(
        dst_feat, feat_1, feat_2, feat_3,
        WT_w_0, WT_b_0.reshape(1, D), WT_w_1, WT_b_1.reshape(1, D),
        WT_w_2, WT_b_2.reshape(1, D), WT_w_3, WT_b_3.reshape(1, D), al, ar)
    ea, sp = _phase1(el.reshape(R * N), er.reshape(R * N), src, dst)
    rinv = _stage_b(sp.reshape(NC, R, N))
    out_part = _phase2(nf1, nf2, nf3, ea, rinv.reshape(R * N), src, dst)
    zm, aw8 = _stage_c1(out_part, sem_W1, sem_b1.reshape(1, D),
                        sem_W2.reshape(1, D))
    return _stage_c2(zm, aw8[0, :R])
